# baseline (speedup 1.0000x reference)
import functools

import jax
import jax.numpy as jnp
from jax.experimental import pallas as pl

_NUM_E = 1024
_DIM = 64
_BETA = 0.25


def _vq_body(x_ref, cb_ref, x2_ref, q_ref, idx_ref, loss_ref):
    b = pl.program_id(0)
    j = pl.program_id(1)
    X = x_ref[0]
    CB = cb_ref[...]
    x2 = x2_ref[0]

    e2 = jnp.sum(CB * CB, axis=1, keepdims=True)

    mm = jax.lax.dot_general(CB, X, (((1,), (0,)), ((), ())))
    d = (x2 + e2) - 2.0 * mm

    m = jnp.min(d, axis=0, keepdims=True)
    eidx = jax.lax.broadcasted_iota(jnp.int32, d.shape, 0)
    idx = jnp.min(jnp.where(d == m, eidx, _NUM_E), axis=0,
                  keepdims=True)
    idx_ref[0] = idx

    E = (eidx == idx).astype(jnp.float32)
    q = jax.lax.dot_general(CB, E, (((0,), (0,)), ((), ())))
    q_ref[0] = q

    diff = q - X
    s = jnp.full((8, 128), jnp.sum(diff * diff), jnp.float32)

    @pl.when(jnp.logical_and(b == 0, j == 0))
    def _init():
        loss_ref[...] = s

    @pl.when(jnp.logical_or(b != 0, j != 0))
    def _acc():
        loss_ref[...] = loss_ref[...] + s


def kernel(inputs, codebook):
    B, C, H, W = inputs.shape
    HW = H * W
    TW = 256
    xr = inputs.reshape(B, C, HW)
    flat = jnp.transpose(inputs, (0, 2, 3, 1)).reshape(-1, C)
    x2 = jnp.sum(flat ** 2, axis=1).reshape(B, 1, HW)

    q, idx, loss_acc = pl.pallas_call(
        _vq_body,
        grid=(B, HW // TW),
        in_specs=[
            pl.BlockSpec((1, C, TW), lambda b, j: (b, 0, j)),
            pl.BlockSpec((_NUM_E, C), lambda b, j: (0, 0)),
            pl.BlockSpec((1, 1, TW), lambda b, j: (b, 0, j)),
        ],
        out_specs=[
            pl.BlockSpec((1, C, TW), lambda b, j: (b, 0, j)),
            pl.BlockSpec((1, 1, TW), lambda b, j: (b, 0, j)),
            pl.BlockSpec((8, 128), lambda b, j: (0, 0)),
        ],
        out_shape=[
            jax.ShapeDtypeStruct((B, C, HW), jnp.float32),
            jax.ShapeDtypeStruct((B, 1, HW), jnp.int32),
            jax.ShapeDtypeStruct((8, 128), jnp.float32),
        ],
    )(xr, codebook, x2)

    quantized_out = q.reshape(B, C, H, W)
    encoding_indices = idx.reshape(B * HW)
    e_latent = loss_acc[0, 0] / (B * HW * C)
    vq_loss = e_latent + _BETA * e_latent
    return quantized_out, vq_loss, encoding_indices


# speedup vs baseline: 1.5322x; 1.5322x over previous
import functools

import jax
import jax.numpy as jnp
from jax.experimental import pallas as pl

_NUM_E = 1024
_DIM = 64
_BETA = 0.25


def _vq_body(x_ref, cb_ref, x2_ref, e2_ref, q_ref, idx_ref, loss_ref):
    b = pl.program_id(0)
    X = x_ref[0]
    CB = cb_ref[...]
    x2 = x2_ref[0]
    e2 = e2_ref[...]

    mm = jax.lax.dot_general(CB, X, (((1,), (0,)), ((), ())))
    d = (x2 + e2) - 2.0 * mm

    m = jnp.min(d, axis=0, keepdims=True)
    eidx = jax.lax.broadcasted_iota(jnp.int32, d.shape, 0)
    idx = jnp.min(jnp.where(d == m, eidx, _NUM_E), axis=0,
                  keepdims=True)
    idx_ref[0] = idx

    E = (eidx == idx).astype(jnp.float32)
    q = jax.lax.dot_general(CB, E, (((0,), (0,)), ((), ())))
    q_ref[0] = q

    diff = q - X
    s = jnp.full((8, 128), jnp.sum(diff * diff), jnp.float32)

    @pl.when(b == 0)
    def _init():
        loss_ref[...] = s

    @pl.when(b != 0)
    def _acc():
        loss_ref[...] = loss_ref[...] + s


def kernel(inputs, codebook):
    B, C, H, W = inputs.shape
    HW = H * W
    xr = inputs.reshape(B, C, HW)
    flat = jnp.transpose(inputs, (0, 2, 3, 1)).reshape(-1, C)
    x2 = jnp.sum(flat ** 2, axis=1).reshape(B, 1, HW)
    e2 = jnp.sum(codebook ** 2, axis=1).reshape(_NUM_E, 1)

    q, idx, loss_acc = pl.pallas_call(
        _vq_body,
        grid=(B,),
        in_specs=[
            pl.BlockSpec((1, C, HW), lambda b: (b, 0, 0)),
            pl.BlockSpec((_NUM_E, C), lambda b: (0, 0)),
            pl.BlockSpec((1, 1, HW), lambda b: (b, 0, 0)),
            pl.BlockSpec((_NUM_E, 1), lambda b: (0, 0)),
        ],
        out_specs=[
            pl.BlockSpec((1, C, HW), lambda b: (b, 0, 0)),
            pl.BlockSpec((1, 1, HW), lambda b: (b, 0, 0)),
            pl.BlockSpec((8, 128), lambda b: (0, 0)),
        ],
        out_shape=[
            jax.ShapeDtypeStruct((B, C, HW), jnp.float32),
            jax.ShapeDtypeStruct((B, 1, HW), jnp.int32),
            jax.ShapeDtypeStruct((8, 128), jnp.float32),
        ],
    )(xr, codebook, x2, e2)

    quantized_out = q.reshape(B, C, H, W)
    encoding_indices = idx.reshape(B * HW)
    e_latent = loss_acc[0, 0] / (B * HW * C)
    vq_loss = e_latent + _BETA * e_latent
    return quantized_out, vq_loss, encoding_indices
